# Initial kernel scaffold; baseline (speedup 1.0000x reference)
import jax
import jax.numpy as jnp
from jax.experimental import pallas as pl


def kernel(hidden_states, gate_weight, gate_up_proj, down_proj, sh_gate_proj, sh_up_proj, sh_down_proj, shared_gate_w):
    raise NotImplementedError("write your pallas kernel here")


# speedup vs baseline: 1.4560x; 1.4560x over previous
import functools

import jax
import jax.numpy as jnp
from jax import lax
from jax.experimental import pallas as pl
from jax.experimental.pallas import tpu as pltpu

B, S, H = 2, 2048, 1024
E, TOPK = 8, 2
I_MOE = 1024
I_SH = 2816
N = B * S
P = N * TOPK

BM = 256
CAP = P + E * BM
NB = CAP // BM

BMR = 512
BMS = 512
BIS = 1408
NIS = I_SH // BIS


def _router_body(x_ref, gw_ref, a1_ref, a2_ref, w1_ref, w2_ref, aux_ref,
                 psum_ref, cnt_ref):
    step = pl.program_id(0)
    x = x_ref[...]
    gw = gw_ref[...]
    logits = lax.dot_general(x, gw, (((1,), (1,)), ((), ())),
                             preferred_element_type=jnp.float32)
    ii = lax.broadcasted_iota(jnp.int32, logits.shape, 1)
    m1 = jnp.max(logits, axis=1, keepdims=True)
    a1 = jnp.min(jnp.where(logits >= m1, ii, E), axis=1, keepdims=True)
    l2 = jnp.where(ii == a1, -jnp.inf, logits)
    m2 = jnp.max(l2, axis=1, keepdims=True)
    a2 = jnp.min(jnp.where(l2 >= m2, ii, E), axis=1, keepdims=True)
    ex = jnp.exp(logits - m1)
    s = jnp.sum(ex, axis=1, keepdims=True)
    w1 = 1.0 / s
    w2 = jnp.exp(m2 - m1) / s
    a1_ref[...] = a1
    a2_ref[...] = a2
    w1_ref[...] = w1
    w2_ref[...] = w2

    probs = ex / s
    onehot = (ii == a1).astype(jnp.float32) + (ii == a2).astype(jnp.float32)

    @pl.when(step == 0)
    def _init():
        psum_ref[...] = jnp.zeros_like(psum_ref)
        cnt_ref[...] = jnp.zeros_like(cnt_ref)

    psum_ref[...] += jnp.sum(probs, axis=0, keepdims=True)
    cnt_ref[...] += jnp.sum(onehot, axis=0, keepdims=True)

    @pl.when(step == pl.num_programs(0) - 1)
    def _fin():
        frac_tok = cnt_ref[...] / float(N * TOPK)
        frac_prob = psum_ref[...] / float(N)
        aux_ref[0, 0] = float(E) * jnp.sum(frac_tok * frac_prob)


def _router(x, gate_weight):
    grid = (N // BMR,)
    return pl.pallas_call(
        _router_body,
        grid=grid,
        in_specs=[
            pl.BlockSpec((BMR, H), lambda i: (i, 0)),
            pl.BlockSpec((E, H), lambda i: (0, 0)),
        ],
        out_specs=[
            pl.BlockSpec((BMR, 1), lambda i: (i, 0)),
            pl.BlockSpec((BMR, 1), lambda i: (i, 0)),
            pl.BlockSpec((BMR, 1), lambda i: (i, 0)),
            pl.BlockSpec((BMR, 1), lambda i: (i, 0)),
            pl.BlockSpec(memory_space=pltpu.SMEM),
        ],
        out_shape=[
            jax.ShapeDtypeStruct((N, 1), jnp.int32),
            jax.ShapeDtypeStruct((N, 1), jnp.int32),
            jax.ShapeDtypeStruct((N, 1), jnp.float32),
            jax.ShapeDtypeStruct((N, 1), jnp.float32),
            jax.ShapeDtypeStruct((1, 1), jnp.float32),
        ],
        scratch_shapes=[
            pltpu.VMEM((1, E), jnp.float32),
            pltpu.VMEM((1, E), jnp.float32),
        ],
    )(x, gate_weight)


def _ffn_body(be_ref, nb_ref, xs_ref, gu_ref, dn_ref, ys_ref):
    b = pl.program_id(0)

    @pl.when(b < nb_ref[0])
    def _():
        x = xs_ref[...]
        gu = gu_ref[0]
        gup = lax.dot_general(x, gu, (((1,), (1,)), ((), ())),
                              preferred_element_type=jnp.float32)
        g = gup[:, :I_MOE]
        u = gup[:, I_MOE:]
        h = g * jax.nn.sigmoid(g) * u
        dn = dn_ref[0]
        ys_ref[...] = lax.dot_general(h, dn, (((1,), (1,)), ((), ())),
                                      preferred_element_type=jnp.float32)


def _grouped_ffn(block_expert, nblocks, xs, gate_up_proj, down_proj):
    grid_spec = pltpu.PrefetchScalarGridSpec(
        num_scalar_prefetch=2,
        grid=(NB,),
        in_specs=[
            pl.BlockSpec((BM, H), lambda b, be, nb: (b, 0)),
            pl.BlockSpec((1, 2 * I_MOE, H), lambda b, be, nb: (be[b], 0, 0)),
            pl.BlockSpec((1, H, I_MOE), lambda b, be, nb: (be[b], 0, 0)),
        ],
        out_specs=pl.BlockSpec((BM, H), lambda b, be, nb: (b, 0)),
    )
    return pl.pallas_call(
        _ffn_body,
        grid_spec=grid_spec,
        out_shape=jax.ShapeDtypeStruct((CAP, H), jnp.float32),
    )(block_expert, nblocks, xs, gate_up_proj, down_proj)


def _shared_body(x_ref, g_ref, u_ref, d_ref, sgw_ref, out_ref):
    i = pl.program_id(1)
    x = x_ref[...]
    gw = g_ref[...]
    uw = u_ref[...]
    g = lax.dot_general(x, gw, (((1,), (1,)), ((), ())),
                        preferred_element_type=jnp.float32)
    u = lax.dot_general(x, uw, (((1,), (1,)), ((), ())),
                        preferred_element_type=jnp.float32)
    h = g * jax.nn.sigmoid(g) * u
    dw = d_ref[...]
    contrib = lax.dot_general(h, dw, (((1,), (1,)), ((), ())),
                              preferred_element_type=jnp.float32)

    @pl.when(i == 0)
    def _():
        out_ref[...] = jnp.zeros_like(out_ref)

    out_ref[...] += contrib

    @pl.when(i == pl.num_programs(1) - 1)
    def _():
        sgw = sgw_ref[...]
        z = lax.dot_general(x, sgw, (((1,), (1,)), ((), ())),
                            preferred_element_type=jnp.float32)
        out_ref[...] *= jax.nn.sigmoid(z)


def _shared_ffn(x, sh_gate, sh_up, sh_down, shared_gate_w):
    grid = (N // BMS, NIS)
    return pl.pallas_call(
        _shared_body,
        grid=grid,
        in_specs=[
            pl.BlockSpec((BMS, H), lambda m, i: (m, 0)),
            pl.BlockSpec((BIS, H), lambda m, i: (i, 0)),
            pl.BlockSpec((BIS, H), lambda m, i: (i, 0)),
            pl.BlockSpec((H, BIS), lambda m, i: (0, i)),
            pl.BlockSpec((1, H), lambda m, i: (0, 0)),
        ],
        out_specs=pl.BlockSpec((BMS, H), lambda m, i: (m, 0)),
        out_shape=jax.ShapeDtypeStruct((N, H), jnp.float32),
    )(x, sh_gate, sh_up, sh_down, shared_gate_w)


def kernel(hidden_states, gate_weight, gate_up_proj, down_proj,
           sh_gate_proj, sh_up_proj, sh_down_proj, shared_gate_w):
    x = hidden_states.reshape(N, H)

    a1, a2, w1, w2, aux = _router(x, gate_weight)
    a1 = a1[:, 0]
    a2 = a2[:, 0]

    eid = jnp.concatenate([a1, a2])
    counts = jnp.sum(
        (eid[:, None] == jnp.arange(E)[None, :]).astype(jnp.int32), axis=0)
    padded = ((counts + BM - 1) // BM) * BM
    coff = jnp.concatenate([jnp.zeros((1,), jnp.int32),
                            jnp.cumsum(counts)]).astype(jnp.int32)
    poff = jnp.concatenate([jnp.zeros((1,), jnp.int32),
                            jnp.cumsum(padded)]).astype(jnp.int32)
    order = jnp.argsort(eid, stable=True)
    ej = eid[order]
    dest = (jnp.arange(P, dtype=jnp.int32) - coff[ej] + poff[ej])
    tok = (order % N).astype(jnp.int32)
    xs = jnp.zeros((CAP, H), jnp.float32).at[dest].set(x[tok])
    pos = jnp.zeros((P,), jnp.int32).at[order].set(dest)
    pos1, pos2 = pos[:N], pos[N:]

    nblocks = (poff[E] // BM).reshape(1)
    bidx = jnp.arange(NB, dtype=jnp.int32) * BM
    block_expert = jnp.sum(
        (poff[1:E + 1][None, :] <= bidx[:, None]).astype(jnp.int32), axis=1)
    block_expert = jnp.minimum(block_expert, E - 1)

    ys = _grouped_ffn(block_expert, nblocks, xs, gate_up_proj, down_proj)
    shared = _shared_ffn(x, sh_gate_proj, sh_up_proj, sh_down_proj,
                         shared_gate_w)

    out = w1 * ys[pos1] + w2 * ys[pos2] + shared
    return out.reshape(B, S, H), aux[0, 0]
